# baseline (speedup 1.0000x reference)
import jax
import jax.numpy as jnp
from jax import lax
from jax.experimental import pallas as pl
from jax.experimental.pallas import tpu as pltpu
from jax.experimental.pallas import tpu_sc as plsc

BATCH = 1024
N_AUGS = 64
N_CLASSES = 1000
N_SUB = 8

_NC = 2
_NS = 16
_NW = _NC * _NS
_ROWS_SC = 600
_ROWS_TC = N_CLASSES - _ROWS_SC
_COLS = 256
_CSPLIT = BATCH // _COLS
_UNITS = (_ROWS_SC // 8) * _CSPLIT
_UPW = -(-_UNITS // _NW)
_NVEC = _COLS // 16
_BR = 200
_TCB = _ROWS_TC // _BR
_BASE = _ROWS_SC // _BR


def _body(xt_hbm, idxs_hbm, out_hbm, idxs_v, bufs, obuf, sems):
    wid = lax.axis_index("s") * _NC + lax.axis_index("c")
    pltpu.sync_copy(idxs_hbm, idxs_v.at[pl.ds(0, N_SUB)])
    ivec = idxs_v[...]
    lanes = lax.iota(jnp.int32, 16)
    a_j = [jnp.sum(jnp.where(lanes == j, ivec, 0)) for j in range(N_SUB)]

    def unit_coords(u):
        row0 = pl.multiple_of((u >> 1) * 8, 8)
        col0 = pl.multiple_of((u & 1) * _COLS, 128)
        return row0, col0

    def issue(u, slot):
        row0, col0 = unit_coords(u)
        for j in range(N_SUB):
            pltpu.async_copy(
                xt_hbm.at[a_j[j], pl.ds(row0, 8), pl.ds(col0, _COLS)],
                bufs.at[slot, j], sems.at[slot])

    def drain(slot):
        pltpu.make_async_copy(
            xt_hbm.at[pl.ds(0, N_SUB), pl.ds(0, 8), pl.ds(0, _COLS)],
            bufs.at[slot], sems.at[slot]).wait()

    issue(wid, 0)
    issue(wid + _NW, 1)
    for t in range(_UPW):
        u = wid + _NW * t
        slot = t % 3
        if t + 2 < _UPW:
            nxt = wid + _NW * (t + 2)

            @pl.when(nxt < _UNITS)
            def _(t=t):
                issue(wid + _NW * (t + 2), (t + 2) % 3)

        @pl.when(u < _UNITS)
        def _(t=t, u=u, slot=slot):
            drain(slot)

            def _red(i, _, slot=slot):
                o = pl.multiple_of(i * 16, 16)
                for r in range(8):
                    vsum = bufs[slot, 0, r, pl.ds(o, 16)]
                    for j in range(1, N_SUB):
                        vsum = vsum + bufs[slot, j, r, pl.ds(o, 16)]
                    obuf[r, pl.ds(o, 16)] = vsum * 0.125
                return 0

            lax.fori_loop(0, _NVEC, _red, 0)
            row0, col0 = unit_coords(u)
            pltpu.sync_copy(obuf,
                            out_hbm.at[pl.ds(row0, 8), pl.ds(col0, _COLS)])


def _tc_body(idxs_ref, x_ref, o_ref):
    j = pl.program_id(1)

    @pl.when(j == 0)
    def _():
        o_ref[...] = x_ref[0] * 0.125

    @pl.when(j > 0)
    def _():
        o_ref[...] = o_ref[...] + x_ref[0] * 0.125


_tc_call = pl.pallas_call(
    _tc_body,
    grid_spec=pltpu.PrefetchScalarGridSpec(
        num_scalar_prefetch=1,
        grid=(_TCB, N_SUB),
        in_specs=[pl.BlockSpec((1, _BR, BATCH),
                               lambda i, j, idxs: (idxs[j], _BASE + i, 0))],
        out_specs=pl.BlockSpec((_BR, BATCH), lambda i, j, idxs: (i, 0)),
    ),
    out_shape=jax.ShapeDtypeStruct((_ROWS_TC, BATCH), jnp.float32),
    compiler_params=pltpu.CompilerParams(
        dimension_semantics=("parallel", "arbitrary")),
)


_sc_call = pl.kernel(
    _body,
    out_type=jax.ShapeDtypeStruct((_ROWS_SC, BATCH), jnp.float32),
    mesh=plsc.VectorSubcoreMesh(core_axis_name="c", subcore_axis_name="s"),
    compiler_params=pltpu.CompilerParams(use_tc_tiling_on_sc=True,
                                         needs_layout_passes=False),
    scratch_types=[
        pltpu.VMEM((16,), jnp.int32),
        pltpu.VMEM((3, N_SUB, 8, _COLS), jnp.float32),
        pltpu.VMEM((8, _COLS), jnp.float32),
        pltpu.SemaphoreType.DMA((3,)),
    ],
)


def kernel(x, idxs):
    xt = jnp.transpose(x, (1, 2, 0))
    i32 = idxs.astype(jnp.int32)
    out_sc = _sc_call(xt, i32)
    out_tc = _tc_call(i32, xt)
    out_t = jnp.concatenate([out_sc, out_tc], axis=0)
    return jnp.transpose(out_t, (1, 0))


# speedup vs baseline: 1.1066x; 1.1066x over previous
import jax
import jax.numpy as jnp
from jax import lax
from jax.experimental import pallas as pl
from jax.experimental.pallas import tpu as pltpu
from jax.experimental.pallas import tpu_sc as plsc

BATCH = 1024
N_AUGS = 64
N_CLASSES = 1000
N_SUB = 8

_NC = 2
_NS = 16
_NW = _NC * _NS
_ROWS_TC = 488
_ROWS_SC = N_CLASSES - _ROWS_TC
_SC_OFF = _ROWS_TC
_COLS = 512
_CSPLIT = BATCH // _COLS
_UNITS = (_ROWS_SC // 8) * _CSPLIT
_UPW = -(-_UNITS // _NW)
_NVEC = _COLS // 16
_BR = _ROWS_TC
_TCB = _ROWS_TC // _BR
_BASE = 0


def _body(xt_hbm, idxs_hbm, out_hbm, idxs_v, bufs, obuf, sems):
    wid = lax.axis_index("s") * _NC + lax.axis_index("c")
    pltpu.sync_copy(idxs_hbm, idxs_v.at[pl.ds(0, N_SUB)])
    ivec = idxs_v[...]
    lanes = lax.iota(jnp.int32, 16)
    a_j = [jnp.sum(jnp.where(lanes == j, ivec, 0)) for j in range(N_SUB)]

    def unit_coords(u):
        row0 = pl.multiple_of(_SC_OFF + (u >> 1) * 8, 8)
        col0 = pl.multiple_of((u & 1) * _COLS, 128)
        return row0, col0

    def issue(u, slot):
        row0, col0 = unit_coords(u)
        for j in range(N_SUB):
            pltpu.async_copy(
                xt_hbm.at[a_j[j], pl.ds(row0, 8), pl.ds(col0, _COLS)],
                bufs.at[slot, j], sems.at[slot])

    def drain(slot):
        pltpu.make_async_copy(
            xt_hbm.at[pl.ds(0, N_SUB), pl.ds(0, 8), pl.ds(0, _COLS)],
            bufs.at[slot], sems.at[slot]).wait()

    issue(wid, 0)
    issue(wid + _NW, 1)
    for t in range(_UPW):
        u = wid + _NW * t
        slot = t % 3
        if t + 2 < _UPW:
            nxt = wid + _NW * (t + 2)

            @pl.when(nxt < _UNITS)
            def _(t=t):
                issue(wid + _NW * (t + 2), (t + 2) % 3)

        @pl.when(u < _UNITS)
        def _(t=t, u=u, slot=slot):
            drain(slot)

            def _red(i, _, slot=slot):
                o = pl.multiple_of(i * 16, 16)
                for r in range(8):
                    vsum = bufs[slot, 0, r, pl.ds(o, 16)]
                    for j in range(1, N_SUB):
                        vsum = vsum + bufs[slot, j, r, pl.ds(o, 16)]
                    obuf[r, pl.ds(o, 16)] = vsum * 0.125
                return 0

            lax.fori_loop(0, _NVEC, _red, 0)
            row0, col0 = unit_coords(u)
            pltpu.sync_copy(obuf,
                            out_hbm.at[pl.ds(row0, 8), pl.ds(col0, _COLS)])


def _tc_body(idxs_ref, x_ref, o_ref):
    j = pl.program_id(1)

    @pl.when(j == 0)
    def _():
        o_ref[...] = x_ref[0] * 0.125

    @pl.when(j > 0)
    def _():
        o_ref[...] = o_ref[...] + x_ref[0] * 0.125


_tc_call = pl.pallas_call(
    _tc_body,
    grid_spec=pltpu.PrefetchScalarGridSpec(
        num_scalar_prefetch=1,
        grid=(_TCB, N_SUB),
        in_specs=[pl.BlockSpec((1, _BR, BATCH),
                               lambda i, j, idxs: (idxs[j], _BASE + i, 0))],
        out_specs=pl.BlockSpec((_BR, BATCH), lambda i, j, idxs: (i, 0)),
    ),
    out_shape=jax.ShapeDtypeStruct((_ROWS_TC, BATCH), jnp.float32),
    compiler_params=pltpu.CompilerParams(
        dimension_semantics=("parallel", "arbitrary")),
)


_sc_call = pl.kernel(
    _body,
    out_type=jax.ShapeDtypeStruct((_ROWS_SC, BATCH), jnp.float32),
    mesh=plsc.VectorSubcoreMesh(core_axis_name="c", subcore_axis_name="s"),
    compiler_params=pltpu.CompilerParams(use_tc_tiling_on_sc=True,
                                         needs_layout_passes=False),
    scratch_types=[
        pltpu.VMEM((16,), jnp.int32),
        pltpu.VMEM((3, N_SUB, 8, _COLS), jnp.float32),
        pltpu.VMEM((8, _COLS), jnp.float32),
        pltpu.SemaphoreType.DMA((3,)),
    ],
)


def kernel(x, idxs):
    xt = jnp.transpose(x, (1, 2, 0))
    i32 = idxs.astype(jnp.int32)
    out_sc = _sc_call(xt, i32)
    out_tc = _tc_call(i32, xt)
    out_t = jnp.concatenate([out_tc, out_sc], axis=0)
    return jnp.transpose(out_t, (1, 0))


# speedup vs baseline: 1.1069x; 1.0002x over previous
import jax
import jax.numpy as jnp
from jax import lax
from jax.experimental import pallas as pl
from jax.experimental.pallas import tpu as pltpu
from jax.experimental.pallas import tpu_sc as plsc

BATCH = 1024
N_AUGS = 64
N_CLASSES = 1000
N_SUB = 8

_NC = 2
_NS = 16
_NW = _NC * _NS
_ROWS_TC = 488
_ROWS_SC = N_CLASSES - _ROWS_TC
_SC_OFF = _ROWS_TC
_COLS = 512
_CSPLIT = BATCH // _COLS
_UNITS = (_ROWS_SC // 8) * _CSPLIT
_UPW = -(-_UNITS // _NW)
_NVEC = _COLS // 16
_BR = _ROWS_TC
_TCB = _ROWS_TC // _BR
_BASE = 0


def _body(xt_hbm, idxs_hbm, out_hbm, idxs_v, bufs, obuf, sems):
    wid = lax.axis_index("s") * _NC + lax.axis_index("c")
    pltpu.sync_copy(idxs_hbm, idxs_v.at[pl.ds(0, N_SUB)])
    ivec = idxs_v[...]
    lanes = lax.iota(jnp.int32, 16)
    a_j = [jnp.sum(jnp.where(lanes == j, ivec, 0)) for j in range(N_SUB)]

    def unit_coords(u):
        row0 = pl.multiple_of((u >> 1) * 8, 8)
        col0 = pl.multiple_of((u & 1) * _COLS, 128)
        return row0, col0

    def issue(u, slot):
        row0, col0 = unit_coords(u)
        for j in range(N_SUB):
            pltpu.async_copy(
                xt_hbm.at[a_j[j], pl.ds(_SC_OFF + row0, 8), pl.ds(col0, _COLS)],
                bufs.at[slot, j], sems.at[slot])

    def drain(slot):
        pltpu.make_async_copy(
            xt_hbm.at[pl.ds(0, N_SUB), pl.ds(0, 8), pl.ds(0, _COLS)],
            bufs.at[slot], sems.at[slot]).wait()

    issue(wid, 0)
    issue(wid + _NW, 1)
    for t in range(_UPW):
        u = wid + _NW * t
        slot = t % 3
        if t + 2 < _UPW:
            nxt = wid + _NW * (t + 2)

            @pl.when(nxt < _UNITS)
            def _(t=t):
                issue(wid + _NW * (t + 2), (t + 2) % 3)

        @pl.when(u < _UNITS)
        def _(t=t, u=u, slot=slot):
            drain(slot)

            def _red(i, _, slot=slot):
                o = pl.multiple_of(i * 16, 16)
                for r in range(8):
                    vsum = bufs[slot, 0, r, pl.ds(o, 16)]
                    for j in range(1, N_SUB):
                        vsum = vsum + bufs[slot, j, r, pl.ds(o, 16)]
                    obuf[r, pl.ds(o, 16)] = vsum * 0.125
                return 0

            lax.fori_loop(0, _NVEC, _red, 0)
            row0, col0 = unit_coords(u)
            pltpu.sync_copy(obuf,
                            out_hbm.at[pl.ds(row0, 8), pl.ds(col0, _COLS)])


def _tc_body(idxs_ref, x_ref, o_ref):
    j = pl.program_id(1)

    @pl.when(j == 0)
    def _():
        o_ref[...] = x_ref[0] * 0.125

    @pl.when(j > 0)
    def _():
        o_ref[...] = o_ref[...] + x_ref[0] * 0.125


_tc_call = pl.pallas_call(
    _tc_body,
    grid_spec=pltpu.PrefetchScalarGridSpec(
        num_scalar_prefetch=1,
        grid=(_TCB, N_SUB),
        in_specs=[pl.BlockSpec((1, _BR, BATCH),
                               lambda i, j, idxs: (idxs[j], _BASE + i, 0))],
        out_specs=pl.BlockSpec((_BR, BATCH), lambda i, j, idxs: (i, 0)),
    ),
    out_shape=jax.ShapeDtypeStruct((_ROWS_TC, BATCH), jnp.float32),
    compiler_params=pltpu.CompilerParams(
        dimension_semantics=("parallel", "arbitrary")),
)


_sc_call = pl.kernel(
    _body,
    out_type=jax.ShapeDtypeStruct((_ROWS_SC, BATCH), jnp.float32),
    mesh=plsc.VectorSubcoreMesh(core_axis_name="c", subcore_axis_name="s"),
    compiler_params=pltpu.CompilerParams(use_tc_tiling_on_sc=True,
                                         needs_layout_passes=False),
    scratch_types=[
        pltpu.VMEM((16,), jnp.int32),
        pltpu.VMEM((3, N_SUB, 8, _COLS), jnp.float32),
        pltpu.VMEM((8, _COLS), jnp.float32),
        pltpu.SemaphoreType.DMA((3,)),
    ],
)


def kernel(x, idxs):
    xt = jnp.transpose(x, (1, 2, 0))
    i32 = idxs.astype(jnp.int32)
    out_sc = _sc_call(xt, i32)
    out_tc = _tc_call(i32, xt)
    out_t = jnp.concatenate([out_tc, out_sc], axis=0)
    return jnp.transpose(out_t, (1, 0))
